# baseline (speedup 1.0000x reference)
import functools
import math

import jax
import jax.numpy as jnp
from jax import lax
from jax.experimental import pallas as pl
from jax.experimental.pallas import tpu as pltpu
from jax.experimental.pallas import tpu_sc as plsc

_HALF_LOG_2PI = 0.5 * math.log(2.0 * math.pi)
_LOG2E = 1.4426950408889634
_LN2 = 0.6931471805599453

_L = 64
_K = 128
_B = 4096
_B_SC = 1024
_NC = 2
_NS = 16
_NW = _NC * _NS
_LPW = _L // _NW
_LANES = 16


def _ln_newton(s, iters=9):
    y = jnp.full((_LANES,), 4.8530, jnp.float32)
    for _ in range(iters):
        y = (y - 1.0) + s * jnp.exp(-y)
    return y


def _xlane_reduce(v, tmp_v, op):
    for sh in (8, 4, 2, 1):
        tmp_v[pl.ds(0, _LANES)] = v
        tmp_v[pl.ds(_LANES, _LANES)] = v
        v = op(v, tmp_v[pl.ds(sh, _LANES)])
    return v


def _sc_body(zt_hbm, mut_hbm, lvt_hbm, w_hbm, out_hbm,
             zt_v, mut_v, lvt_v, w_v, abc_v, o_v, tmp_v, abc_sm):
    wid = lax.axis_index("s") * _NC + lax.axis_index("c")
    l0 = wid * _LPW
    pltpu.sync_copy(zt_hbm.at[pl.ds(l0, _LPW)], zt_v)
    pltpu.sync_copy(mut_hbm.at[pl.ds(l0 * _K, _LPW * _K)], mut_v)
    pltpu.sync_copy(lvt_hbm.at[pl.ds(l0 * _K, _LPW * _K)], lvt_v)
    pltpu.sync_copy(w_hbm, w_v)

    nkc = _K // _LANES

    wmv = jnp.full((_LANES,), -1e30, jnp.float32)
    for kc in range(nkc):
        wmv = jnp.maximum(wmv, w_v[pl.ds(kc * _LANES, _LANES)])
    wmax = _xlane_reduce(wmv, tmp_v, jnp.maximum)
    sacc = jnp.zeros((_LANES,), jnp.float32)
    for kc in range(nkc):
        sacc = sacc + jnp.exp(w_v[pl.ds(kc * _LANES, _LANES)] - wmax)
    s_w = _xlane_reduce(sacc, tmp_v, jnp.add)
    neg_norm = -(wmax + _ln_newton(s_w))

    for l in range(_LPW):
        for kc in range(nkc):
            src = pl.ds(l * _K + kc * _LANES, _LANES)
            muv = mut_v[src]
            lvv = lvt_v[src]
            lwv = w_v[pl.ds(kc * _LANES, _LANES)]
            pv = jnp.exp(-lvv)
            base = 3 * l * _K + kc * _LANES
            abc_v[pl.ds(base, _LANES)] = ((-_HALF_LOG_2PI) - 0.5 * lvv
                                          - 0.5 * pv * muv * muv + lwv)
            abc_v[pl.ds(base + _K, _LANES)] = pv * muv
            abc_v[pl.ds(base + 2 * _K, _LANES)] = -0.5 * pv
    for i in range(3 * _LPW * _K // _LANES):
        vv = abc_v[pl.ds(i * _LANES, _LANES)]
        for j in range(_LANES):
            abc_sm[i * _LANES + j] = vv[j]

    GC = 8
    nblk = _B_SC // (GC * _LANES)

    for l in range(_LPW):
        aoff = 3 * l * _K

        def blk(i, _, l=l, aoff=aoff):
            boff = i * GC * _LANES
            zv = [zt_v[l, pl.ds(boff + bc * _LANES, _LANES)]
                  for bc in range(GC)]
            z2 = [v * v for v in zv]

            def pass1(k, m):
                a = abc_sm[aoff + k]
                b = abc_sm[aoff + _K + k]
                c = abc_sm[aoff + 2 * _K + k]
                return tuple(
                    jnp.maximum(m[bc], a + b * zv[bc] + c * z2[bc])
                    for bc in range(GC))

            m0 = tuple(jnp.full((_LANES,), -1e30, jnp.float32)
                       for _x in range(GC))
            m = lax.fori_loop(0, _K, pass1, m0, unroll=2)

            def pass2(k, s):
                a = abc_sm[aoff + k]
                b = abc_sm[aoff + _K + k]
                c = abc_sm[aoff + 2 * _K + k]
                return tuple(
                    s[bc] + jnp.exp(a + b * zv[bc] + c * z2[bc] - m[bc])
                    for bc in range(GC))

            s0 = tuple(jnp.zeros((_LANES,), jnp.float32)
                       for _x in range(GC))
            s = lax.fori_loop(0, _K, pass2, s0, unroll=2)
            for bc in range(GC):
                o_v[l, pl.ds(boff + bc * _LANES, _LANES)] = (
                    m[bc] + _ln_newton(s[bc]) + neg_norm)
            return _

        lax.fori_loop(0, nblk, blk, 0)

    pltpu.sync_copy(o_v, out_hbm.at[pl.ds(l0, _LPW)])


_sc_call = functools.partial(
    pl.kernel,
    out_type=jax.ShapeDtypeStruct((_L, _B_SC), jnp.float32),
    mesh=plsc.VectorSubcoreMesh(core_axis_name="c", subcore_axis_name="s"),
    scratch_types=[
        pltpu.VMEM((_LPW, _B_SC), jnp.float32),
        pltpu.VMEM((_LPW * _K,), jnp.float32),
        pltpu.VMEM((_LPW * _K,), jnp.float32),
        pltpu.VMEM((_K,), jnp.float32),
        pltpu.VMEM((3 * _LPW * _K,), jnp.float32),
        pltpu.VMEM((_LPW, _B_SC), jnp.float32),
        pltpu.VMEM((2 * _LANES,), jnp.float32),
        pltpu.SMEM((3 * _LPW * _K,), jnp.float32),
    ],
)(_sc_body)


_LG = 8
_KC = 32


def _tc_body(zt_ref, mu_ref, lv_ref, w_ref, out_ref, p3_s):
    K, L = mu_ref.shape
    Bb = zt_ref.shape[1]
    NG = L // _LG
    mu = mu_ref[...]
    lv = lv_ref[...]
    wv = w_ref[...]
    wmax = jnp.max(wv)
    logw = wv - wmax - jnp.log(jnp.sum(jnp.exp(wv - wmax)))
    prec = jnp.exp(-lv)
    a_all = _LOG2E * ((-_HALF_LOG_2PI) - 0.5 * lv
                      - 0.5 * prec * mu * mu + logw)
    b_all = _LOG2E * prec * mu
    c_all = (-0.5 * _LOG2E) * prec
    for g in range(NG):
        sl = slice(g * _LG, (g + 1) * _LG)
        p3_s[pl.ds(g, 1)] = jnp.concatenate(
            [a_all[:, sl], b_all[:, sl], c_all[:, sl]], axis=0)[None]

    def lgroup(g, _):
        pc = p3_s[pl.ds(g, 1)][0]
        for j in range(_LG):
            col = pc[:, j:j + 1]
            a = col[0:K]
            b = col[K:2 * K]
            c = col[2 * K:3 * K]
            zrow = zt_ref[pl.ds(g * _LG + j, 1), :]
            z2 = zrow * zrow
            m_run = None
            s_run = None
            for kc in range(K // _KC):
                ks = slice(kc * _KC, (kc + 1) * _KC)
                t2c = a[ks] + b[ks] * zrow + c[ks] * z2
                t3 = t2c.reshape(_KC // 8, 8, Bb)
                mc = jnp.max(t3, axis=0)
                sc = jnp.sum(jnp.exp2(t3 - mc[None]), axis=0)
                if m_run is None:
                    m_run, s_run = mc, sc
                else:
                    m_new = jnp.maximum(m_run, mc)
                    s_run = (s_run * jnp.exp2(m_run - m_new)
                             + sc * jnp.exp2(mc - m_new))
                    m_run = m_new
            m1 = jnp.max(m_run, axis=0, keepdims=True)
            s1 = jnp.sum(s_run * jnp.exp2(m_run - m1),
                         axis=0, keepdims=True)
            out_ref[pl.ds(g * _LG + j, 1), :] = _LN2 * (m1 + jnp.log2(s1))
        return 0

    lax.fori_loop(0, NG, lgroup, 0)


def _tc_call(zt, means, logvars, w2):
    L, Btc = zt.shape
    K = means.shape[0]
    Bb = 1024
    grid = (Btc // Bb,)
    return pl.pallas_call(
        _tc_body,
        grid=grid,
        in_specs=[
            pl.BlockSpec((L, Bb), lambda i: (0, i)),
            pl.BlockSpec((K, L), lambda i: (0, 0)),
            pl.BlockSpec((K, L), lambda i: (0, 0)),
            pl.BlockSpec((K, 1), lambda i: (0, 0)),
        ],
        out_specs=pl.BlockSpec((L, Bb), lambda i: (0, i)),
        out_shape=jax.ShapeDtypeStruct((L, Btc), jnp.float32),
        scratch_shapes=[
            pltpu.VMEM((L // _LG, 3 * K, _LG), jnp.float32),
        ],
    )(zt, means, logvars, w2)


@jax.jit
def kernel(z, means, logvars, w):
    B, L = z.shape
    K = means.shape[0]
    zt = z.T
    w2 = w.reshape(K, 1)
    out_sc = _sc_call(zt[:, :_B_SC], means.T.reshape(-1),
                      logvars.T.reshape(-1), w.reshape(K))
    out_tc = _tc_call(zt[:, _B_SC:], means, logvars, w2)
    return jnp.concatenate([out_sc, out_tc], axis=1).T


# speedup vs baseline: 1.1264x; 1.1264x over previous
import functools
import math

import jax
import jax.numpy as jnp
from jax import lax
from jax.experimental import pallas as pl
from jax.experimental.pallas import tpu as pltpu
from jax.experimental.pallas import tpu_sc as plsc

_HALF_LOG_2PI = 0.5 * math.log(2.0 * math.pi)
_LOG2E = 1.4426950408889634
_LN2 = 0.6931471805599453

_L = 64
_K = 128
_B = 4096
_B_SC = 1024
_NC = 2
_NS = 16
_NW = _NC * _NS
_LPW = _L // _NW
_LANES = 16


def _ln_newton(s, iters=2):
    bits = lax.bitcast_convert_type(s, jnp.int32).astype(jnp.float32)
    y = bits * (_LN2 / 8388608.0) - ((127.0 - 0.043) * _LN2)
    for _ in range(iters):
        y = (y - 1.0) + s * jnp.exp(-y)
    return y


def _xlane_reduce(v, tmp_v, op):
    for sh in (8, 4, 2, 1):
        tmp_v[pl.ds(0, _LANES)] = v
        tmp_v[pl.ds(_LANES, _LANES)] = v
        v = op(v, tmp_v[pl.ds(sh, _LANES)])
    return v


def _sc_body(zt_hbm, mut_hbm, lvt_hbm, w_hbm, out_hbm,
             zt_v, mut_v, lvt_v, w_v, abc_v, o_v, tmp_v, abc_sm):
    wid = lax.axis_index("s") * _NC + lax.axis_index("c")
    l0 = wid * _LPW
    pltpu.sync_copy(zt_hbm.at[pl.ds(l0, _LPW)], zt_v)
    pltpu.sync_copy(mut_hbm.at[pl.ds(l0 * _K, _LPW * _K)], mut_v)
    pltpu.sync_copy(lvt_hbm.at[pl.ds(l0 * _K, _LPW * _K)], lvt_v)
    pltpu.sync_copy(w_hbm, w_v)

    nkc = _K // _LANES

    wmv = jnp.full((_LANES,), -1e30, jnp.float32)
    for kc in range(nkc):
        wmv = jnp.maximum(wmv, w_v[pl.ds(kc * _LANES, _LANES)])
    wmax = _xlane_reduce(wmv, tmp_v, jnp.maximum)
    sacc = jnp.zeros((_LANES,), jnp.float32)
    for kc in range(nkc):
        sacc = sacc + jnp.exp(w_v[pl.ds(kc * _LANES, _LANES)] - wmax)
    s_w = _xlane_reduce(sacc, tmp_v, jnp.add)
    neg_norm = -(wmax + _ln_newton(s_w))

    for l in range(_LPW):
        for kc in range(nkc):
            src = pl.ds(l * _K + kc * _LANES, _LANES)
            muv = mut_v[src]
            lvv = lvt_v[src]
            lwv = w_v[pl.ds(kc * _LANES, _LANES)]
            pv = jnp.exp(-lvv)
            base = 3 * l * _K + kc * _LANES
            abc_v[pl.ds(base, _LANES)] = ((-_HALF_LOG_2PI) - 0.5 * lvv
                                          - 0.5 * pv * muv * muv + lwv)
            abc_v[pl.ds(base + _K, _LANES)] = pv * muv
            abc_v[pl.ds(base + 2 * _K, _LANES)] = -0.5 * pv
    for i in range(3 * _LPW * _K // _LANES):
        vv = abc_v[pl.ds(i * _LANES, _LANES)]
        for j in range(_LANES):
            abc_sm[i * _LANES + j] = vv[j]

    GC = 8
    nblk = _B_SC // (GC * _LANES)

    for l in range(_LPW):
        aoff = 3 * l * _K

        def blk(i, _, l=l, aoff=aoff):
            boff = i * GC * _LANES
            zv = [zt_v[l, pl.ds(boff + bc * _LANES, _LANES)]
                  for bc in range(GC)]
            z2 = [v * v for v in zv]

            def pass1(k, m):
                a = abc_sm[aoff + k]
                b = abc_sm[aoff + _K + k]
                c = abc_sm[aoff + 2 * _K + k]
                return tuple(
                    jnp.maximum(m[bc], a + b * zv[bc] + c * z2[bc])
                    for bc in range(GC))

            m0 = tuple(jnp.full((_LANES,), -1e30, jnp.float32)
                       for _x in range(GC))
            m = lax.fori_loop(0, _K, pass1, m0, unroll=2)

            def pass2(k, s):
                a = abc_sm[aoff + k]
                b = abc_sm[aoff + _K + k]
                c = abc_sm[aoff + 2 * _K + k]
                return tuple(
                    s[bc] + jnp.exp(a + b * zv[bc] + c * z2[bc] - m[bc])
                    for bc in range(GC))

            s0 = tuple(jnp.zeros((_LANES,), jnp.float32)
                       for _x in range(GC))
            s = lax.fori_loop(0, _K, pass2, s0, unroll=2)
            for bc in range(GC):
                o_v[l, pl.ds(boff + bc * _LANES, _LANES)] = (
                    m[bc] + _ln_newton(s[bc]) + neg_norm)
            return _

        lax.fori_loop(0, nblk, blk, 0)

    pltpu.sync_copy(o_v, out_hbm.at[pl.ds(l0, _LPW)])


_sc_call = functools.partial(
    pl.kernel,
    out_type=jax.ShapeDtypeStruct((_L, _B_SC), jnp.float32),
    mesh=plsc.VectorSubcoreMesh(core_axis_name="c", subcore_axis_name="s"),
    scratch_types=[
        pltpu.VMEM((_LPW, _B_SC), jnp.float32),
        pltpu.VMEM((_LPW * _K,), jnp.float32),
        pltpu.VMEM((_LPW * _K,), jnp.float32),
        pltpu.VMEM((_K,), jnp.float32),
        pltpu.VMEM((3 * _LPW * _K,), jnp.float32),
        pltpu.VMEM((_LPW, _B_SC), jnp.float32),
        pltpu.VMEM((2 * _LANES,), jnp.float32),
        pltpu.SMEM((3 * _LPW * _K,), jnp.float32),
    ],
)(_sc_body)


_LG = 8
_KC = 32


def _tc_body(zt_ref, mu_ref, lv_ref, w_ref, out_ref, p3_s):
    K, L = mu_ref.shape
    Bb = zt_ref.shape[1]
    NG = L // _LG
    mu = mu_ref[...]
    lv = lv_ref[...]
    wv = w_ref[...]
    wmax = jnp.max(wv)
    logw = wv - wmax - jnp.log(jnp.sum(jnp.exp(wv - wmax)))
    prec = jnp.exp(-lv)
    a_all = _LOG2E * ((-_HALF_LOG_2PI) - 0.5 * lv
                      - 0.5 * prec * mu * mu + logw)
    b_all = _LOG2E * prec * mu
    c_all = (-0.5 * _LOG2E) * prec
    for g in range(NG):
        sl = slice(g * _LG, (g + 1) * _LG)
        p3_s[pl.ds(g, 1)] = jnp.concatenate(
            [a_all[:, sl], b_all[:, sl], c_all[:, sl]], axis=0)[None]

    def lgroup(g, _):
        pc = p3_s[pl.ds(g, 1)][0]
        for j in range(_LG):
            col = pc[:, j:j + 1]
            a = col[0:K]
            b = col[K:2 * K]
            c = col[2 * K:3 * K]
            zrow = zt_ref[pl.ds(g * _LG + j, 1), :]
            z2 = zrow * zrow
            m_run = None
            s_run = None
            for kc in range(K // _KC):
                ks = slice(kc * _KC, (kc + 1) * _KC)
                t2c = a[ks] + b[ks] * zrow + c[ks] * z2
                t3 = t2c.reshape(_KC // 8, 8, Bb)
                mc = jnp.max(t3, axis=0)
                sc = jnp.sum(jnp.exp2(t3 - mc[None]), axis=0)
                if m_run is None:
                    m_run, s_run = mc, sc
                else:
                    m_new = jnp.maximum(m_run, mc)
                    s_run = (s_run * jnp.exp2(m_run - m_new)
                             + sc * jnp.exp2(mc - m_new))
                    m_run = m_new
            m1 = jnp.max(m_run, axis=0, keepdims=True)
            s1 = jnp.sum(s_run * jnp.exp2(m_run - m1),
                         axis=0, keepdims=True)
            out_ref[pl.ds(g * _LG + j, 1), :] = _LN2 * (m1 + jnp.log2(s1))
        return 0

    lax.fori_loop(0, NG, lgroup, 0)


def _tc_call(zt, means, logvars, w2):
    L, Btc = zt.shape
    K = means.shape[0]
    Bb = 512
    grid = (Btc // Bb,)
    return pl.pallas_call(
        _tc_body,
        grid=grid,
        in_specs=[
            pl.BlockSpec((L, Bb), lambda i: (0, i)),
            pl.BlockSpec((K, L), lambda i: (0, 0)),
            pl.BlockSpec((K, L), lambda i: (0, 0)),
            pl.BlockSpec((K, 1), lambda i: (0, 0)),
        ],
        out_specs=pl.BlockSpec((L, Bb), lambda i: (0, i)),
        out_shape=jax.ShapeDtypeStruct((L, Btc), jnp.float32),
        scratch_shapes=[
            pltpu.VMEM((L // _LG, 3 * K, _LG), jnp.float32),
        ],
    )(zt, means, logvars, w2)


@jax.jit
def kernel(z, means, logvars, w):
    B, L = z.shape
    K = means.shape[0]
    zt = z.T
    w2 = w.reshape(K, 1)
    out_sc = _sc_call(zt[:, :_B_SC], means.T.reshape(-1),
                      logvars.T.reshape(-1), w.reshape(K))
    out_tc = _tc_call(zt[:, _B_SC:], means, logvars, w2)
    return jnp.concatenate([out_sc, out_tc], axis=1).T
